# baseline (speedup 1.0000x reference)
import functools

import jax
import jax.numpy as jnp
from jax import lax
from jax.experimental import pallas as pl
from jax.experimental.pallas import tpu as pltpu
from jax.experimental.pallas import tpu_sc as plsc

_L = 16
_NW = 32
_IDX_CHUNK = 128


def _make_sc_loss(B, D):
    b_per_w = B // _NW
    n_chunk = b_per_w // _IDX_CHUNK
    vecs_per_row = D // _L
    mesh = plsc.VectorSubcoreMesh(core_axis_name="c", subcore_axis_name="s")

    @functools.partial(
        pl.kernel,
        mesh=mesh,
        out_type=jax.ShapeDtypeStruct((_NW, _L), jnp.float32),
        scratch_types=[
            pltpu.VMEM((n_chunk, _IDX_CHUNK), jnp.int32),
            pltpu.VMEM((b_per_w, D), jnp.float32),
            pltpu.VMEM((b_per_w, D), jnp.float32),
            pltpu.VMEM((_L,), jnp.float32),
            pltpu.SemaphoreType.DMA,
        ],
        compiler_params=pltpu.CompilerParams(use_tc_tiling_on_sc=False),
    )
    def sc_loss(tgt_hbm, emb_hbm, cent_hbm, out_hbm, idx_v, rows_v, emb_v,
                acc_v, sem):
        wid = lax.axis_index("s") * 2 + lax.axis_index("c")
        base = wid * b_per_w
        pltpu.sync_copy(tgt_hbm.at[wid], idx_v)
        copies = [
            pltpu.async_copy(
                cent_hbm.at[idx_v.at[g]],
                rows_v.at[pl.ds(g * _IDX_CHUNK, _IDX_CHUNK)],
                sem,
            )
            for g in range(n_chunk)
        ]
        pltpu.sync_copy(emb_hbm.at[pl.ds(base, b_per_w)], emb_v)
        for cp in copies:
            cp.wait()

        zero = jnp.zeros((_L,), jnp.float32)

        def body(i, accs):
            out = []
            for j in range(vecs_per_row):
                e = emb_v[i, pl.ds(j * _L, _L)]
                c = rows_v[i, pl.ds(j * _L, _L)]
                d = e - c
                out.append(accs[j] + d * d)
            return tuple(out)

        accs = lax.fori_loop(0, b_per_w, body, (zero,) * vecs_per_row)
        total = accs[0]
        for j in range(1, vecs_per_row):
            total = total + accs[j]
        acc_v[...] = total
        pltpu.sync_copy(acc_v, out_hbm.at[wid])

    return sc_loss


def kernel(target, vector_embedding, centers):
    B, D = vector_embedding.shape
    tgt = target.astype(jnp.int32).reshape(_NW, -1, _IDX_CHUNK)
    partials = _make_sc_loss(B, D)(tgt, vector_embedding, centers)
    return jnp.sum(partials) * (0.5 / B)
